# baseline (speedup 1.0000x reference)
import jax
import jax.numpy as jnp
from jax import lax
from jax.experimental import pallas as pl
from jax.experimental.pallas import tpu as pltpu
from jax.experimental.pallas import tpu_sc as plsc

_MAGIC = 12582912.0

N_ROWS = 1048576
NW = 32
ROWS_PER_W = N_ROWS // NW
CHUNK = 256
NCHUNK = ROWS_PER_W // CHUNK
GROUPS = CHUNK // 16


def _rne(x):
    return (x + _MAGIC) - _MAGIC


def _decode8(zs):
    fs, ds, absds = [], [], []
    sum_f = sum_d2 = m = None
    for z in zs:
        f = _rne(z)
        d = z - f
        a = jnp.abs(d)
        d2 = d * d
        sum_f = f if sum_f is None else sum_f + f
        sum_d2 = d2 if sum_d2 is None else sum_d2 + d2
        m = a if m is None else jnp.maximum(m, a)
        fs.append(f)
        ds.append(d)
        absds.append(a)
    h = sum_f * 0.5
    odd = _rne(h) != h
    dist = sum_d2 + jnp.where(odd, 1.0 - (m + m), 0.0)
    modd = jnp.where(odd, m, -1.0)
    gs = []
    for f, d, a in zip(fs, ds, absds):
        stp = jnp.where(d >= 0, 1.0, -1.0)
        gs.append(f + jnp.where(a == modd, stp, 0.0))
    return gs, dist


def _sc_kernel(x_hbm, o_hbm, v):
    cid = lax.axis_index("c")
    sid = lax.axis_index("s")
    wid = sid * 2 + cid
    iota16 = lax.iota(jnp.int32, 16)
    cols = [jnp.full((16,), j, jnp.int32) for j in range(8)]

    def chunk_body(c, carry):
        base = wid * ROWS_PER_W + c * CHUNK
        pltpu.sync_copy(x_hbm.at[pl.ds(base, CHUNK)], v)

        def body(g, carry2):
            ridx = g * 16 + iota16
            xs = [plsc.load_gather(v, [ridx, cols[j]]) for j in range(8)]
            g0s, d0 = _decode8(xs)
            g1s, d1 = _decode8([x - 0.5 for x in xs])
            ch = d0 <= d1
            for j in range(8):
                y = jnp.where(ch, g0s[j], g1s[j] + 0.5)
                plsc.store_scatter(v, [ridx, cols[j]], y)
            return carry2

        lax.fori_loop(0, GROUPS, body, 0)
        pltpu.sync_copy(v, o_hbm.at[pl.ds(base, CHUNK)])
        return carry

    lax.fori_loop(0, NCHUNK, chunk_body, 0)


@jax.jit
def _e8_sc(x):
    mesh = plsc.VectorSubcoreMesh(core_axis_name="c", subcore_axis_name="s")
    f = pl.kernel(
        _sc_kernel,
        mesh=mesh,
        out_type=jax.ShapeDtypeStruct((N_ROWS, 8), jnp.float32),
        scratch_types=[
            pltpu.VMEM((CHUNK, 8), jnp.float32),
        ],
        compiler_params=pltpu.CompilerParams(needs_layout_passes=False),
    )
    return f(x)


def kernel(x):
    return _e8_sc(x)


# speedup vs baseline: 1.0438x; 1.0438x over previous
import jax
import jax.numpy as jnp
from jax import lax
from jax.experimental import pallas as pl
from jax.experimental.pallas import tpu as pltpu
from jax.experimental.pallas import tpu_sc as plsc

_MAGIC = 12582912.0

N_ROWS = 1048576
NW = 32
ROWS_PER_W = N_ROWS // NW
CHUNK = 512
NCHUNK = ROWS_PER_W // CHUNK
GROUPS = CHUNK // 16


def _rne(x):
    return (x + _MAGIC) - _MAGIC


def _decode8(zs):
    fs, ds, absds = [], [], []
    sum_f = sum_d2 = m = None
    for z in zs:
        f = _rne(z)
        d = z - f
        a = jnp.abs(d)
        d2 = d * d
        sum_f = f if sum_f is None else sum_f + f
        sum_d2 = d2 if sum_d2 is None else sum_d2 + d2
        m = a if m is None else jnp.maximum(m, a)
        fs.append(f)
        ds.append(d)
        absds.append(a)
    h = sum_f * 0.5
    odd = _rne(h) != h
    dist = sum_d2 + jnp.where(odd, 1.0 - (m + m), 0.0)
    modd = jnp.where(odd, m, -1.0)
    gs = []
    for f, d, a in zip(fs, ds, absds):
        stp = jnp.where(d >= 0, 1.0, -1.0)
        gs.append(f + jnp.where(a == modd, stp, 0.0))
    return gs, dist


def _sc_kernel(x_hbm, o_hbm, v):
    cid = lax.axis_index("c")
    sid = lax.axis_index("s")
    wid = sid * 2 + cid
    iota16 = lax.iota(jnp.int32, 16)
    cols = [jnp.full((16,), j, jnp.int32) for j in range(8)]

    def chunk_body(c, carry):
        base = wid * ROWS_PER_W + c * CHUNK
        pltpu.sync_copy(x_hbm.at[pl.ds(base, CHUNK)], v)

        def body(g, carry2):
            ridx = g * 16 + iota16
            xs = [plsc.load_gather(v, [ridx, cols[j]]) for j in range(8)]
            g0s, d0 = _decode8(xs)
            g1s, d1 = _decode8([x - 0.5 for x in xs])
            ch = d0 <= d1
            for j in range(8):
                y = jnp.where(ch, g0s[j], g1s[j] + 0.5)
                plsc.store_scatter(v, [ridx, cols[j]], y)
            return carry2

        lax.fori_loop(0, GROUPS, body, 0)
        pltpu.sync_copy(v, o_hbm.at[pl.ds(base, CHUNK)])
        return carry

    lax.fori_loop(0, NCHUNK, chunk_body, 0)


@jax.jit
def _e8_sc(x):
    mesh = plsc.VectorSubcoreMesh(core_axis_name="c", subcore_axis_name="s")
    f = pl.kernel(
        _sc_kernel,
        mesh=mesh,
        out_type=jax.ShapeDtypeStruct((N_ROWS, 8), jnp.float32),
        scratch_types=[
            pltpu.VMEM((CHUNK, 8), jnp.float32),
        ],
        compiler_params=pltpu.CompilerParams(needs_layout_passes=False),
    )
    return f(x)


def kernel(x):
    return _e8_sc(x)


# speedup vs baseline: 1.4301x; 1.3700x over previous
import functools

import jax
import jax.numpy as jnp
from jax.experimental import pallas as pl
from jax.experimental.pallas import tpu as pltpu


def _decode_d8_t(z, iota8):
    f = jnp.round(z)
    delta = z - f
    absd = jnp.abs(delta)
    m = jnp.max(absd, axis=0, keepdims=True)
    ki = jnp.min(jnp.where(absd >= m, iota8, jnp.int32(8)), axis=0,
                 keepdims=True)
    is_k = iota8 == ki
    s = jnp.sum(f, axis=0, keepdims=True)
    h = s * jnp.float32(0.5)
    odd = jnp.round(h) != h
    stp = jnp.where(delta >= 0, jnp.float32(1.0), jnp.float32(-1.0))
    g = f + jnp.where(jnp.logical_and(is_k, odd), stp, jnp.float32(0.0))
    d2 = jnp.sum(delta * delta, axis=0, keepdims=True)
    d = d2 + jnp.where(odd, jnp.float32(1.0) - (m + m), jnp.float32(0.0))
    return g, d


def _e8_body(x_ref, o_ref):
    xt = jnp.transpose(x_ref[...])
    iota8 = jax.lax.broadcasted_iota(jnp.int32, xt.shape, 0)
    g0, d0 = _decode_d8_t(xt, iota8)
    g1, d1 = _decode_d8_t(xt - jnp.float32(0.5), iota8)
    g1 = g1 + jnp.float32(0.5)
    y = jnp.where(d0 <= d1, g0, g1)
    o_ref[...] = jnp.transpose(y)


@functools.partial(jax.jit, static_argnames=("block_rows",))
def _e8_tc(x, block_rows=8192):
    n = x.shape[0]
    grid = n // block_rows
    return pl.pallas_call(
        _e8_body,
        grid=(grid,),
        in_specs=[pl.BlockSpec((block_rows, 8), lambda i: (i, 0))],
        out_specs=pl.BlockSpec((block_rows, 8), lambda i: (i, 0)),
        out_shape=jax.ShapeDtypeStruct(x.shape, x.dtype),
    )(x)


def kernel(x):
    return _e8_tc(x)


# speedup vs baseline: 12.3387x; 8.6277x over previous
import functools

import jax
import jax.numpy as jnp
from jax.experimental import pallas as pl


def _decode_d8_t(z, iota8):
    f = jnp.round(z)
    delta = z - f
    absd = jnp.abs(delta)
    m = jnp.max(absd, axis=0, keepdims=True)
    ki = jnp.min(jnp.where(absd >= m, iota8, jnp.int32(8)), axis=0,
                 keepdims=True)
    is_k = iota8 == ki
    s = jnp.sum(f, axis=0, keepdims=True)
    h = s * jnp.float32(0.5)
    odd = jnp.round(h) != h
    stp = jnp.where(delta >= 0, jnp.float32(1.0), jnp.float32(-1.0))
    g = f + jnp.where(jnp.logical_and(is_k, odd), stp, jnp.float32(0.0))
    d2 = jnp.sum(delta * delta, axis=0, keepdims=True)
    d = d2 + jnp.where(odd, jnp.float32(1.0) - (m + m), jnp.float32(0.0))
    return g, d


def _e8_body(x_ref, o_ref):
    xt = x_ref[...]
    iota8 = jax.lax.broadcasted_iota(jnp.int32, xt.shape, 0)
    g0, d0 = _decode_d8_t(xt, iota8)
    g1, d1 = _decode_d8_t(xt - jnp.float32(0.5), iota8)
    g1 = g1 + jnp.float32(0.5)
    o_ref[...] = jnp.where(d0 <= d1, g0, g1)


@functools.partial(jax.jit, static_argnames=("block",))
def _e8_tc(x, block=32768):
    xt = jnp.transpose(x)
    n = xt.shape[1]
    grid = n // block
    yt = pl.pallas_call(
        _e8_body,
        grid=(grid,),
        in_specs=[pl.BlockSpec((8, block), lambda i: (0, i))],
        out_specs=pl.BlockSpec((8, block), lambda i: (0, i)),
        out_shape=jax.ShapeDtypeStruct(xt.shape, x.dtype),
    )(xt)
    return jnp.transpose(yt)


def kernel(x):
    return _e8_tc(x)


# speedup vs baseline: 16.6337x; 1.3481x over previous
import functools

import jax
import jax.numpy as jnp
from jax.experimental import pallas as pl


def _decode_d8_t(z):
    f = jnp.round(z)
    delta = z - f
    absd = jnp.abs(delta)
    m = jnp.max(absd, axis=0, keepdims=True)
    s = jnp.sum(f, axis=0, keepdims=True)
    h = s * jnp.float32(0.5)
    odd = jnp.round(h) != h
    mb = jnp.where(odd, m, jnp.float32(-1.0))
    stp = jnp.where(delta >= 0, jnp.float32(1.0), jnp.float32(-1.0))
    g = f + jnp.where(absd == mb, stp, jnp.float32(0.0))
    dadj = jnp.where(odd, jnp.float32(1.0) - (m + m), jnp.float32(0.0))
    return g, delta, dadj


def _e8_body(x_ref, o_ref):
    xt = x_ref[...]
    g0, delta0, adj0 = _decode_d8_t(xt)
    g1, delta1, adj1 = _decode_d8_t(xt - jnp.float32(0.5))
    dd = jnp.sum(delta0 * delta0 - delta1 * delta1, axis=0, keepdims=True)
    dd = dd + (adj0 - adj1)
    o_ref[...] = jnp.where(dd <= 0, g0, g1 + jnp.float32(0.5))


@functools.partial(jax.jit, static_argnames=("block",))
def _e8_tc(x, block=32768):
    xt = jnp.transpose(x)
    n = xt.shape[1]
    grid = n // block
    yt = pl.pallas_call(
        _e8_body,
        grid=(grid,),
        in_specs=[pl.BlockSpec((8, block), lambda i: (0, i))],
        out_specs=pl.BlockSpec((8, block), lambda i: (0, i)),
        out_shape=jax.ShapeDtypeStruct(xt.shape, x.dtype),
    )(xt)
    return jnp.transpose(yt)


def kernel(x):
    return _e8_tc(x)
